# baseline (speedup 1.0000x reference)
import jax
import jax.numpy as jnp
from jax import lax
from jax.experimental import pallas as pl
from jax.experimental.pallas import tpu as pltpu
from jax.experimental.pallas import tpu_sc as plsc

N = 10000
NP = 10240
E = 320000
D = 128
NC = 2
NS = 16
NW = NC * NS
EPT = E // NW
G = 20
NG = EPT // G
NB = 6
NQ = 12
RPT = NP // NS
ZR = 64
MAIN = (NG - 8) // NQ


def _sc_body(x_hbm, idx_hbm, e_hbm, out_hbm, acc, idxr,
             xr0, xr1, xr2, xr3, xr4, xr5,
             er0, er1, er2, er3, er4, er5, zbuf,
             gsem, esem, ssem, isem):
    c = lax.axis_index("c")
    s = lax.axis_index("s")
    wid = s * NC + c
    XR = (xr0, xr1, xr2, xr3, xr4, xr5)
    ER = (er0, er1, er2, er3, er4, er5)
    zero = jnp.zeros((16,), jnp.float32)

    def issue_idx(g, q):
        pltpu.async_copy(idx_hbm.at[wid, g], idxr.at[q], isem.at[q])

    def wait_idx(q):
        pltpu.make_async_copy(idx_hbm.at[wid, 0], idxr.at[q], isem.at[q]).wait()

    def issue_gather(q, b):
        pltpu.async_copy(x_hbm.at[idxr.at[q, 0]], XR[b], gsem.at[b])

    def wait_gather(q, b):
        pltpu.make_async_copy(x_hbm.at[idxr.at[q, 0]], XR[b], gsem.at[b]).wait()

    def issue_e(g, b):
        pltpu.async_copy(e_hbm.at[wid, g], ER[b], esem.at[b])

    def wait_e(b):
        pltpu.make_async_copy(e_hbm.at[wid, 0], ER[b], esem.at[b]).wait()

    def issue_scatter(q, b):
        pltpu.async_copy(XR[b], acc.at[idxr.at[q, 1]], ssem.at[b], add=True)

    def wait_scatter(q, b):
        pltpu.make_async_copy(XR[b], acc.at[idxr.at[q, 1]], ssem.at[b]).wait()

    def compute(b):
        xr, er = XR[b], ER[b]

        def _row(r, carry):
            for j in range(D // 16):
                sl = pl.ds(j * 16, 16)
                xr[r, sl] = jnp.maximum(xr[r, sl] + er[r, sl], 0.0)
            return carry
        lax.fori_loop(0, G, _row, None)

    def step(g, k, wait_next_idx=True, issue_next=True, issue_idx6=True):
        b = k % NB
        q = k % NQ
        b3 = (k + 3) % NB
        q3 = (k + 3) % NQ
        q6 = (k + 6) % NQ
        wait_gather(q, b)
        wait_e(b)
        if wait_next_idx:
            wait_idx(q3)
        wait_scatter((k + 9) % NQ, b3)
        if issue_next:
            issue_gather(q3, b3)
            issue_e(g + 3, b3)
        if issue_idx6:
            issue_idx(g + 6, q6)
        compute(b)
        issue_scatter(q, b)

    for q in range(6):
        issue_idx(q, q)
    for i in range(3):
        wait_idx(i)
        issue_gather(i, i)
        issue_e(i, i)

    def _zrow(r, carry):
        for j in range(D // 16):
            sl = pl.ds(j * 16, 16)
            zbuf[r, sl] = zero
        return carry
    lax.fori_loop(0, ZR, _zrow, None)

    def _zrow2(r, carry):
        for j in range(D // 16):
            sl = pl.ds(j * 16, 16)
            er3[r, sl] = zero
            er4[r, sl] = zero
            er5[r, sl] = zero
        return carry
    lax.fori_loop(0, G, _zrow2, None)
    for k in range(RPT // ZR):
        pltpu.sync_copy(zbuf, acc.at[pl.ds(s * RPT + k * ZR, ZR)])
    plsc.subcore_barrier()

    pltpu.async_copy(er3, acc.at[idxr.at[0, 1]], ssem.at[3], add=True)
    pltpu.async_copy(er4, acc.at[idxr.at[0, 1]], ssem.at[4], add=True)
    pltpu.async_copy(er5, acc.at[idxr.at[0, 1]], ssem.at[5], add=True)

    def _main(t, carry):
        g = t * NQ
        for k in range(NQ):
            step(g + k, k)
        return carry
    lax.fori_loop(0, MAIN, _main, None)

    g0 = MAIN * NQ
    for g in range(g0, NG):
        step(g, g % NQ,
             wait_next_idx=(g + 3 <= NG - 1),
             issue_next=(g + 3 <= NG - 1),
             issue_idx6=(g + 6 <= NG - 1))
    for g in (NG - 3, NG - 2, NG - 1):
        wait_scatter(g % NQ, g % NB)

    plsc.subcore_barrier()
    pltpu.sync_copy(acc.at[pl.ds(s * RPT, RPT)],
                    out_hbm.at[c, pl.ds(s * RPT, RPT)])


def _sc_partials(x, idx4, e4):
    mesh = plsc.VectorSubcoreMesh(core_axis_name="c", subcore_axis_name="s")
    return pl.kernel(
        _sc_body,
        out_type=jax.ShapeDtypeStruct((NC, NP, D), jnp.float32),
        mesh=mesh,
        scratch_types=[
            pltpu.VMEM_SHARED((NP, D), jnp.float32),
            pltpu.VMEM((NQ, 2, G), jnp.int32),
            pltpu.VMEM((G, D), jnp.float32),
            pltpu.VMEM((G, D), jnp.float32),
            pltpu.VMEM((G, D), jnp.float32),
            pltpu.VMEM((G, D), jnp.float32),
            pltpu.VMEM((G, D), jnp.float32),
            pltpu.VMEM((G, D), jnp.float32),
            pltpu.VMEM((G, D), jnp.float32),
            pltpu.VMEM((G, D), jnp.float32),
            pltpu.VMEM((G, D), jnp.float32),
            pltpu.VMEM((G, D), jnp.float32),
            pltpu.VMEM((G, D), jnp.float32),
            pltpu.VMEM((G, D), jnp.float32),
            pltpu.VMEM((ZR, D), jnp.float32),
            pltpu.SemaphoreType.DMA((NB,)),
            pltpu.SemaphoreType.DMA((NB,)),
            pltpu.SemaphoreType.DMA((NB,)),
            pltpu.SemaphoreType.DMA((NQ,)),
        ],
    )(x, idx4, e4)


def _combine_body(x_ref, p_ref, o_ref):
    o_ref[...] = x_ref[...] + p_ref[0] + p_ref[1]


def _combine(x, partials):
    blk = 1000
    return pl.pallas_call(
        _combine_body,
        out_shape=jax.ShapeDtypeStruct((N, D), jnp.float32),
        grid=(N // blk,),
        in_specs=[
            pl.BlockSpec((blk, D), lambda i: (i, 0)),
            pl.BlockSpec((NC, blk, D), lambda i: (0, i, 0)),
        ],
        out_specs=pl.BlockSpec((blk, D), lambda i: (i, 0)),
    )(x, partials)


@jax.jit
def kernel(node_inputs, edge_index, edge_inputs):
    idx4 = edge_index.reshape(2, NW, NG, G).transpose(1, 2, 0, 3)
    e4 = edge_inputs.reshape(NW, NG, G, D)
    partials = _sc_partials(node_inputs, idx4, e4)
    return _combine(node_inputs, partials)


# speedup vs baseline: 1.8185x; 1.8185x over previous
import jax
import jax.numpy as jnp
from jax import lax
from jax.experimental import pallas as pl
from jax.experimental.pallas import tpu as pltpu
from jax.experimental.pallas import tpu_sc as plsc

N = 10000
NP = 10240
E = 320000
D = 128
NC = 2
NS = 16
NW = NC * NS
EPT = E // NW
G = 40
NG = EPT // G
NB = 4
NQ = 8
RPT = NP // NS
MAIN = (NG - 10) // NQ


def _sc_body(x_hbm, idx_hbm, e_hbm, out_hbm, acc, idxr,
             xr0, xr1, xr2, xr3, er0, er1, er2, er3,
             gsem, esem, ssem, isem):
    c = lax.axis_index("c")
    s = lax.axis_index("s")
    wid = s * NC + c
    XR = (xr0, xr1, xr2, xr3)
    ER = (er0, er1, er2, er3)
    zero = jnp.zeros((16,), jnp.float32)

    def issue_idx(g, q):
        pltpu.async_copy(idx_hbm.at[wid, g], idxr.at[q], isem.at[q])

    def wait_idx(q):
        pltpu.make_async_copy(idx_hbm.at[wid, 0], idxr.at[q], isem.at[q]).wait()

    def issue_gather(q, b):
        pltpu.async_copy(x_hbm.at[idxr.at[q, 0]], XR[b], gsem.at[b])

    def wait_gather(q, b):
        pltpu.make_async_copy(x_hbm.at[idxr.at[q, 0]], XR[b], gsem.at[b]).wait()

    def issue_e(g, b):
        pltpu.async_copy(e_hbm.at[wid, g], ER[b], esem.at[b])

    def wait_e(b):
        pltpu.make_async_copy(e_hbm.at[wid, 0], ER[b], esem.at[b]).wait()

    def issue_scatter(q, b):
        pltpu.async_copy(XR[b], acc.at[idxr.at[q, 1]], ssem.at[b], add=True)

    def wait_scatter(q, b):
        pltpu.make_async_copy(XR[b], acc.at[idxr.at[q, 1]], ssem.at[b]).wait()

    def compute(b):
        xr, er = XR[b], ER[b]

        def _row(r, carry):
            for j in range(D // 16):
                sl = pl.ds(j * 16, 16)
                xr[r, sl] = jnp.maximum(xr[r, sl] + er[r, sl], 0.0)
            return carry
        lax.fori_loop(0, G, _row, None)

    def step(g, k, wait_next_idx=True, issue_next=True, issue_idx4=True):
        b = k % NB
        q = k % NQ
        b2 = (k + 2) % NB
        q2 = (k + 2) % NQ
        q4 = (k + 4) % NQ
        wait_gather(q, b)
        wait_e(b)
        if wait_next_idx:
            wait_idx(q2)
        wait_scatter((k + 6) % NQ, b2)
        if issue_next:
            issue_gather(q2, b2)
            issue_e(g + 2, b2)
        if issue_idx4:
            issue_idx(g + 4, q4)
        compute(b)
        issue_scatter(q, b)

    for q in range(4):
        issue_idx(q, q)
    wait_idx(0)
    issue_gather(0, 0)
    issue_e(0, 0)
    wait_idx(1)
    issue_gather(1, 1)
    issue_e(1, 1)

    def _zrow(r, carry):
        for j in range(D // 16):
            er2[r, pl.ds(j * 16, 16)] = zero
            er3[r, pl.ds(j * 16, 16)] = zero
        return carry
    lax.fori_loop(0, G, _zrow, None)
    for k in range(RPT // G):
        pltpu.sync_copy(ER[2 + (k % 2)], acc.at[pl.ds(s * RPT + k * G, G)])
    plsc.subcore_barrier()

    pltpu.async_copy(er2, acc.at[idxr.at[0, 1]], ssem.at[2], add=True)
    pltpu.async_copy(er3, acc.at[idxr.at[0, 1]], ssem.at[3], add=True)

    def _main(t, carry):
        g = t * NQ
        for k in range(NQ):
            step(g + k, k)
        return carry
    lax.fori_loop(0, MAIN, _main, None)

    g0 = MAIN * NQ
    for g in range(g0, NG):
        step(g, g % NQ,
             wait_next_idx=(g + 2 <= NG - 1),
             issue_next=(g + 2 <= NG - 1),
             issue_idx4=(g + 4 <= NG - 1))
    wait_scatter((NG - 2) % NQ, (NG - 2) % NB)
    wait_scatter((NG - 1) % NQ, (NG - 1) % NB)

    plsc.subcore_barrier()
    pltpu.sync_copy(acc.at[pl.ds(s * RPT, RPT)],
                    out_hbm.at[c, pl.ds(s * RPT, RPT)])


def _sc_partials(x, idx4, e4):
    mesh = plsc.VectorSubcoreMesh(core_axis_name="c", subcore_axis_name="s")
    return pl.kernel(
        _sc_body,
        out_type=jax.ShapeDtypeStruct((NC, NP, D), jnp.float32),
        mesh=mesh,
        scratch_types=[
            pltpu.VMEM_SHARED((NP, D), jnp.float32),
            pltpu.VMEM((NQ, 2, G), jnp.int32),
            pltpu.VMEM((G, D), jnp.float32),
            pltpu.VMEM((G, D), jnp.float32),
            pltpu.VMEM((G, D), jnp.float32),
            pltpu.VMEM((G, D), jnp.float32),
            pltpu.VMEM((G, D), jnp.float32),
            pltpu.VMEM((G, D), jnp.float32),
            pltpu.VMEM((G, D), jnp.float32),
            pltpu.VMEM((G, D), jnp.float32),
            pltpu.SemaphoreType.DMA((NB,)),
            pltpu.SemaphoreType.DMA((NB,)),
            pltpu.SemaphoreType.DMA((NB,)),
            pltpu.SemaphoreType.DMA((NQ,)),
        ],
    )(x, idx4, e4)


def _combine_body(x_ref, p_ref, o_ref):
    o_ref[...] = x_ref[...] + p_ref[0] + p_ref[1]


def _combine(x, partials):
    blk = 1000
    return pl.pallas_call(
        _combine_body,
        out_shape=jax.ShapeDtypeStruct((N, D), jnp.float32),
        grid=(N // blk,),
        in_specs=[
            pl.BlockSpec((blk, D), lambda i: (i, 0)),
            pl.BlockSpec((NC, blk, D), lambda i: (0, i, 0)),
        ],
        out_specs=pl.BlockSpec((blk, D), lambda i: (i, 0)),
    )(x, partials)


@jax.jit
def kernel(node_inputs, edge_index, edge_inputs):
    idx4 = edge_index.reshape(2, NW, NG, G).transpose(1, 2, 0, 3)
    e4 = edge_inputs.reshape(NW, NG, G, D)
    partials = _sc_partials(node_inputs, idx4, e4)
    return _combine(node_inputs, partials)
